# baseline (speedup 1.0000x reference)
import functools

import jax
import jax.numpy as jnp
from jax import lax
from jax.experimental import pallas as pl
from jax.experimental.pallas import tpu as pltpu
from jax.experimental.pallas import tpu_sc as plsc

DIM = 3
D1 = DIM + 1
N = 32768
M = 32768
K = 9
CF = 64
CIN = CF + D1
CH = 40
NHALF = 2
BLK = 2048
CHUNK = 128
NWORKERS = 32

_f32 = jnp.float32


def _lrelu(x):
    return jnp.where(x > 0, x, 0.1 * x)


def _sc_mesh():
    return plsc.VectorSubcoreMesh(core_axis_name="c", subcore_axis_name="s")


_SC_PARAMS = pltpu.CompilerParams(use_tc_tiling_on_sc=False)


def _t1(pts, el_t, bary_t, W1, b1r, W2, b2r, W3, b3r):
    def body(p_ref, el_ref, ba_ref, w1, b1_, w2, b2_, w3, b3_, out_ref):
        cdims = (((1,), (1,)), ((), ()))
        h = _lrelu(lax.dot_general(p_ref[...], w1[...], cdims) + b1_[...])
        h = _lrelu(lax.dot_general(h, w2[...], cdims) + b2_[...])
        h = _lrelu(lax.dot_general(h, w3[...], cdims) + b3_[...])
        half0 = jnp.concatenate([el_ref[...], h[:, : CH - D1]], axis=1)
        half1 = jnp.concatenate(
            [h[:, CH - D1 :], jnp.zeros((BLK, 2 * CH - D1 - CF), _f32)], axis=1
        )
        for hh, half in ((0, half0), (1, half1)):
            for d in range(D1):
                out_ref[hh, d] = half * ba_ref[:, d : d + 1]

    full = lambda a: pl.BlockSpec(a.shape, lambda i: (0,) * a.ndim)
    return pl.pallas_call(
        body,
        grid=(N // BLK,),
        in_specs=[
            pl.BlockSpec((BLK, DIM), lambda i: (i, 0)),
            pl.BlockSpec((BLK, D1), lambda i: (i, 0)),
            pl.BlockSpec((BLK, D1), lambda i: (i, 0)),
            full(W1), full(b1r), full(W2), full(b2r), full(W3), full(b3r),
        ],
        out_specs=pl.BlockSpec((NHALF, D1, BLK, CH), lambda i: (0, 0, i, 0)),
        out_shape=jax.ShapeDtypeStruct((NHALF, D1, N, CH), _f32),
    )(pts, el_t, bary_t, W1, b1r, W2, b2r, W3, b3r)


GRP = 512


def _splat(w1, w2, i1, i2, zeros_stripe):
    upt = (D1 * N) // 16
    ng = upt // GRP
    gch = GRP // CHUNK
    stripe = M // 16

    @functools.partial(
        pl.kernel,
        out_type=(jax.ShapeDtypeStruct((NHALF, M, CH), _f32),) * 2,
        mesh=_sc_mesh(),
        compiler_params=_SC_PARAMS,
        scratch_types=[
            pltpu.VMEM((2, GRP, CH), _f32),
            pltpu.VMEM((2, gch, CHUNK), jnp.int32),
            pltpu.SemaphoreType.DMA,
            pltpu.SemaphoreType.DMA,
            pltpu.VMEM_SHARED((M, CH), _f32),
        ],
    )
    def k(w1_hbm, w2_hbm, i1_hbm, i2_hbm, z_hbm, lat1_hbm, lat2_hbm,
          upd_v, idx_v, sem_in, sem_add, lat_sh):
        cid = lax.axis_index("c")
        sid = lax.axis_index("s")
        for w_hbm, i_hbm, lat_hbm in ((w1_hbm, i1_hbm, lat1_hbm),
                                      (w2_hbm, i2_hbm, lat2_hbm)):
            pltpu.sync_copy(z_hbm, lat_sh.at[pl.ds(sid * stripe, stripe)])
            plsc.subcore_barrier()

            def fire_in(g, b, w_hbm=w_hbm, i_hbm=i_hbm):
                j0 = sid * upt + g * GRP
                dw = pltpu.async_copy(w_hbm.at[cid, pl.ds(j0, GRP)],
                                      upd_v.at[b], sem_in)
                di = pltpu.async_copy(
                    i_hbm.at[pl.ds(sid * (upt // CHUNK) + g * gch, gch)],
                    idx_v.at[b], sem_in)
                return (dw, di)

            def fire_adds(b):
                return [
                    pltpu.async_copy(upd_v.at[b, pl.ds(s * CHUNK, CHUNK)],
                                     lat_sh.at[idx_v.at[b, s]],
                                     sem_add, add=True)
                    for s in range(gch)
                ]

            pend_in = fire_in(0, 0)
            pend_adds = [None, None]
            for g in range(ng):
                b = g % 2
                for d in pend_in:
                    d.wait()
                if g + 1 < ng:
                    if pend_adds[1 - b] is not None:
                        for d in pend_adds[1 - b]:
                            d.wait()
                        pend_adds[1 - b] = None
                    pend_in = fire_in(g + 1, 1 - b)
                pend_adds[b] = fire_adds(b)
            for pa in pend_adds:
                if pa is not None:
                    for d in pa:
                        d.wait()
            plsc.subcore_barrier()
            pltpu.sync_copy(
                lat_sh.at[pl.ds(sid * stripe, stripe)],
                lat_hbm.at[cid, pl.ds(sid * stripe, stripe)],
            )
            plsc.subcore_barrier()

    return k(w1, w2, i1, i2, zeros_stripe)


def _t3(lat_half, Wb1p):
    def body(lat_ref, w_ref, out_ref):
        latb = jnp.concatenate([lat_ref[0], lat_ref[1]], axis=1)
        for k in range(K):
            out_ref[k] = lax.dot_general(
                latb, w_ref[k], (((1,), (1,)), ((), ()))
            )

    return pl.pallas_call(
        body,
        grid=(M // BLK,),
        in_specs=[
            pl.BlockSpec((NHALF, BLK, CH), lambda i: (0, i, 0)),
            pl.BlockSpec(Wb1p.shape, lambda i: (0, 0, 0)),
        ],
        out_specs=pl.BlockSpec((K, BLK, CF), lambda i: (0, i, 0)),
        out_shape=jax.ShapeDtypeStruct((K, M, CF), _f32),
    )(lat_half, Wb1p)


def _blur_gather(p1, p2, bl1, bl2):
    sites = M // NWORKERS
    nch = sites // CHUNK

    @functools.partial(
        pl.kernel,
        out_type=(jax.ShapeDtypeStruct((M, CF), _f32),) * 2,
        mesh=_sc_mesh(),
        compiler_params=_SC_PARAMS,
        scratch_types=[
            pltpu.VMEM((K, CHUNK), jnp.int32),
            pltpu.VMEM((K, CHUNK, CF), _f32),
            pltpu.VMEM((CHUNK, CF), _f32),
            pltpu.SemaphoreType.DMA,
        ],
    )
    def k(p1_hbm, p2_hbm, b1_hbm, b2_hbm, o1_hbm, o2_hbm,
          idx_v, g_v, acc_v, sem):
        cid = lax.axis_index("c")
        sid = lax.axis_index("s")
        wid = sid * NHALF + cid

        for p_hbm, blur_hbm, out_hbm in ((p1_hbm, b1_hbm, o1_hbm),
                                         (p2_hbm, b2_hbm, o2_hbm)):
            def chunk(c, carry, p_hbm=p_hbm, blur_hbm=blur_hbm,
                      out_hbm=out_hbm):
                m0 = wid * sites + c * CHUNK
                pltpu.sync_copy(blur_hbm.at[:, pl.ds(m0, CHUNK)], idx_v)
                descs = [
                    pltpu.async_copy(p_hbm.at[idx_v.at[k_]], g_v.at[k_], sem)
                    for k_ in range(K)
                ]
                for d in descs:
                    d.wait()

                def row(i, carry2):
                    for v in range(CF // 16):
                        sl = pl.ds(v * 16, 16)
                        a = g_v[0, i, sl]
                        for k_ in range(1, K):
                            a = a + g_v[k_, i, sl]
                        acc_v[i, sl] = a
                    return carry2

                lax.fori_loop(0, CHUNK, row, 0)
                pltpu.sync_copy(acc_v, out_hbm.at[pl.ds(m0, CHUNK)])
                return carry

            lax.fori_loop(0, nch, chunk, 0)

    return k(p1, p2, bl1, bl2)


def _t5(hpre, Wb2, bb1r, bb2r):
    def body(h_ref, w_ref, b1_, b2_, out_ref):
        h = _lrelu(h_ref[...] + b1_[...])
        out_ref[...] = _lrelu(
            lax.dot_general(h, w_ref[...], (((1,), (1,)), ((), ()))) + b2_[...]
        )

    full = lambda a: pl.BlockSpec(a.shape, lambda i: (0,) * a.ndim)
    return pl.pallas_call(
        body,
        grid=(M // BLK,),
        in_specs=[
            pl.BlockSpec((BLK, CF), lambda i: (i, 0)),
            full(Wb2), full(bb1r), full(bb2r),
        ],
        out_specs=pl.BlockSpec((BLK, CF), lambda i: (i, 0)),
        out_shape=jax.ShapeDtypeStruct((M, CF), _f32),
    )(hpre, Wb2, bb1r, bb2r)


def _slice_gather(h21, h22, off1, off2):
    pts = N // NWORKERS
    nch = pts // CHUNK

    @functools.partial(
        pl.kernel,
        out_type=(jax.ShapeDtypeStruct((D1, N, CF), _f32),) * 2,
        mesh=_sc_mesh(),
        compiler_params=_SC_PARAMS,
        scratch_types=[
            pltpu.VMEM((D1, CHUNK), jnp.int32),
            pltpu.VMEM((D1, CHUNK, CF), _f32),
            pltpu.SemaphoreType.DMA,
        ],
    )
    def k(h1_hbm, h2_hbm, f1_hbm, f2_hbm, o1_hbm, o2_hbm, idx_v, g_v, sem):
        cid = lax.axis_index("c")
        sid = lax.axis_index("s")
        wid = sid * NHALF + cid

        for h_hbm, off_hbm, out_hbm in ((h1_hbm, f1_hbm, o1_hbm),
                                        (h2_hbm, f2_hbm, o2_hbm)):
            def chunk(c, carry, h_hbm=h_hbm, off_hbm=off_hbm,
                      out_hbm=out_hbm):
                n0 = wid * pts + c * CHUNK
                pltpu.sync_copy(off_hbm.at[:, pl.ds(n0, CHUNK)], idx_v)
                descs = [
                    pltpu.async_copy(h_hbm.at[idx_v.at[d]], g_v.at[d], sem)
                    for d in range(D1)
                ]
                for d in descs:
                    d.wait()
                for d in range(D1):
                    pltpu.sync_copy(g_v.at[d], out_hbm.at[d, pl.ds(n0, CHUNK)])
                return carry

            lax.fori_loop(0, nch, chunk, 0)

    return k(h21, h22, off1, off2)


def _t7(G1, G2, bary1_t, bary2_t, Wout, boutr):
    def body(g1_ref, g2_ref, b1_ref, b2_ref, w_ref, bo_ref, out_ref):
        s1 = g1_ref[0] * b1_ref[:, 0:1]
        s2 = g2_ref[0] * b2_ref[:, 0:1]
        for d in range(1, D1):
            s1 = s1 + g1_ref[d] * b1_ref[:, d : d + 1]
            s2 = s2 + g2_ref[d] * b2_ref[:, d : d + 1]
        fused = jnp.concatenate([s1, s2], axis=1)
        out_ref[...] = (
            lax.dot_general(w_ref[...], fused, (((1,), (1,)), ((), ())))
            + bo_ref[...]
        )

    full = lambda a: pl.BlockSpec(a.shape, lambda i: (0,) * a.ndim)
    return pl.pallas_call(
        body,
        grid=(N // BLK,),
        in_specs=[
            pl.BlockSpec((D1, BLK, CF), lambda i: (0, i, 0)),
            pl.BlockSpec((D1, BLK, CF), lambda i: (0, i, 0)),
            pl.BlockSpec((BLK, D1), lambda i: (i, 0)),
            pl.BlockSpec((BLK, D1), lambda i: (i, 0)),
            full(Wout), full(boutr),
        ],
        out_specs=pl.BlockSpec((3, BLK), lambda i: (0, i)),
        out_shape=jax.ShapeDtypeStruct((3, N), _f32),
    )(G1, G2, bary1_t, bary2_t, Wout, boutr)


def kernel(pc1, pc2, pc1_el_minus_gr, pc2_el_minus_gr, pc1_barycentric,
           pc2_barycentric, pc1_lattice_offset, pc2_lattice_offset,
           pc1_blur_neighbors, pc2_blur_neighbors,
           W1, b1, W2, b2, W3, b3, Wb1, bb1, Wb2, bb2, Wout, bout):
    b1r, b2r, b3r = b1[None], b2[None], b3[None]
    bb1r, bb2r = bb1[None], bb2[None]
    boutr = bout[:, None]
    Wb1p = jnp.transpose(Wb1, (2, 0, 1))
    Wb1p = jnp.pad(Wb1p, ((0, 0), (0, 0), (0, 2 * CH - CIN)))
    zeros_stripe = jnp.zeros((M // 16, CH), _f32)

    def prep(pc, el, bary, offs, blur):
        pts_t = jnp.transpose(pc[0])
        el_t = jnp.transpose(el[0])
        bary_t = jnp.transpose(bary[0])
        idx2d = offs[0]
        idx_rows = idx2d.reshape(D1 * N // CHUNK, CHUNK)
        blur2d = blur[0] + (jnp.arange(K, dtype=jnp.int32) * M)[:, None]
        w_upd = _t1(pts_t, el_t, bary_t, W1, b1r, W2, b2r, W3, b3r)
        w_upd = w_upd.reshape(NHALF, D1 * N, CH)
        return w_upd, idx_rows, idx2d, blur2d, bary_t

    w1u, i1, off1, bl1, bary1_t = prep(pc1, pc1_el_minus_gr, pc1_barycentric,
                                       pc1_lattice_offset, pc1_blur_neighbors)
    w2u, i2, off2, bl2, bary2_t = prep(pc2, pc2_el_minus_gr, pc2_barycentric,
                                       pc2_lattice_offset, pc2_blur_neighbors)
    lat1, lat2 = _splat(w1u, w2u, i1, i2, zeros_stripe)
    p1 = _t3(lat1, Wb1p).reshape(K * M, CF)
    p2 = _t3(lat2, Wb1p).reshape(K * M, CF)
    hpre1, hpre2 = _blur_gather(p1, p2, bl1, bl2)
    h21 = _t5(hpre1, Wb2, bb1r, bb2r)
    h22 = _t5(hpre2, Wb2, bb1r, bb2r)
    g1, g2 = _slice_gather(h21, h22, off1, off2)
    flow = _t7(g1, g2, bary1_t, bary2_t, Wout, boutr)
    return flow[None]


# speedup vs baseline: 1.4012x; 1.4012x over previous
import functools

import jax
import jax.numpy as jnp
from jax import lax
from jax.experimental import pallas as pl
from jax.experimental.pallas import tpu as pltpu
from jax.experimental.pallas import tpu_sc as plsc

DIM = 3
D1 = DIM + 1
N = 32768
M = 32768
K = 9
CF = 64
CIN = CF + D1
CH = 40
NHALF = 2
BLK = 2048
CHUNK = 128
GRP = 512
NWORKERS = 32

_f32 = jnp.float32


def _lrelu(x):
    return jnp.where(x > 0, x, 0.1 * x)


def _sc_mesh():
    return plsc.VectorSubcoreMesh(core_axis_name="c", subcore_axis_name="s")


_SC_UNTILED = pltpu.CompilerParams(use_tc_tiling_on_sc=False)
_SC_TILED = pltpu.CompilerParams(use_tc_tiling_on_sc=True)


def _t1(pts, el_t, bary_t, W1, b1r, W2, b2r, W3, b3r):
    def body(p_ref, el_ref, ba_ref, w1, b1_, w2, b2_, w3, b3_, out_ref):
        cdims = (((1,), (1,)), ((), ()))
        h = _lrelu(lax.dot_general(p_ref[...], w1[...], cdims) + b1_[...])
        h = _lrelu(lax.dot_general(h, w2[...], cdims) + b2_[...])
        h = _lrelu(lax.dot_general(h, w3[...], cdims) + b3_[...])
        half0 = jnp.concatenate([el_ref[...], h[:, : CH - D1]], axis=1)
        half1 = jnp.concatenate(
            [h[:, CH - D1 :], jnp.zeros((BLK, 2 * CH - D1 - CF), _f32)], axis=1
        )
        for hh, half in ((0, half0), (1, half1)):
            for d in range(D1):
                out_ref[hh, d] = half * ba_ref[:, d : d + 1]

    full = lambda a: pl.BlockSpec(a.shape, lambda i: (0,) * a.ndim)
    return pl.pallas_call(
        body,
        grid=(N // BLK,),
        in_specs=[
            pl.BlockSpec((BLK, DIM), lambda i: (i, 0)),
            pl.BlockSpec((BLK, D1), lambda i: (i, 0)),
            pl.BlockSpec((BLK, D1), lambda i: (i, 0)),
            full(W1), full(b1r), full(W2), full(b2r), full(W3), full(b3r),
        ],
        out_specs=pl.BlockSpec((NHALF, D1, BLK, CH), lambda i: (0, 0, i, 0)),
        out_shape=jax.ShapeDtypeStruct((NHALF, D1, N, CH), _f32),
    )(pts, el_t, bary_t, W1, b1r, W2, b2r, W3, b3r)


def _splat(w_upd, idx_rows, zeros_stripe):
    upt = (D1 * N) // 16
    ng = upt // GRP
    gch = GRP // CHUNK
    stripe = M // 16

    @functools.partial(
        pl.kernel,
        out_type=jax.ShapeDtypeStruct((NHALF, M, CH), _f32),
        mesh=_sc_mesh(),
        compiler_params=_SC_UNTILED,
        scratch_types=[
            pltpu.VMEM((2, GRP, CH), _f32),
            pltpu.VMEM((2, gch, CHUNK), jnp.int32),
            pltpu.SemaphoreType.DMA,
            pltpu.SemaphoreType.DMA,
            pltpu.VMEM_SHARED((M, CH), _f32),
        ],
    )
    def k(w_hbm, i_hbm, z_hbm, lat_hbm, upd_v, idx_v, sem_in, sem_add, lat_sh):
        cid = lax.axis_index("c")
        sid = lax.axis_index("s")
        pltpu.sync_copy(z_hbm, lat_sh.at[pl.ds(sid * stripe, stripe)])
        plsc.subcore_barrier()

        def fire_in(g, b):
            j0 = sid * upt + g * GRP
            dw = pltpu.async_copy(w_hbm.at[cid, pl.ds(j0, GRP)],
                                  upd_v.at[b], sem_in)
            di = pltpu.async_copy(
                i_hbm.at[pl.ds(sid * (upt // CHUNK) + g * gch, gch)],
                idx_v.at[b], sem_in)
            return (dw, di)

        def fire_adds(b):
            return [
                pltpu.async_copy(upd_v.at[b, pl.ds(s * CHUNK, CHUNK)],
                                 lat_sh.at[idx_v.at[b, s]],
                                 sem_add, add=True)
                for s in range(gch)
            ]

        pend_in = fire_in(0, 0)
        pend_adds = [None, None]
        for g in range(ng):
            b = g % 2
            for d in pend_in:
                d.wait()
            if g + 1 < ng:
                if pend_adds[1 - b] is not None:
                    for d in pend_adds[1 - b]:
                        d.wait()
                    pend_adds[1 - b] = None
                pend_in = fire_in(g + 1, 1 - b)
            pend_adds[b] = fire_adds(b)
        for pa in pend_adds:
            if pa is not None:
                for d in pa:
                    d.wait()
        plsc.subcore_barrier()
        pltpu.sync_copy(
            lat_sh.at[pl.ds(sid * stripe, stripe)],
            lat_hbm.at[cid, pl.ds(sid * stripe, stripe)],
        )

    return k(w_upd, idx_rows, zeros_stripe)


def _t3(lat_half, Wb1dup):
    def body(lat_ref, w_ref, out_ref):
        latb = jnp.concatenate([lat_ref[0], lat_ref[1]], axis=1)
        for k in range(K):
            out_ref[k] = lax.dot_general(
                latb, w_ref[k], (((1,), (0,)), ((), ()))
            )

    return pl.pallas_call(
        body,
        grid=(M // BLK,),
        in_specs=[
            pl.BlockSpec((NHALF, BLK, CH), lambda i: (0, i, 0)),
            pl.BlockSpec(Wb1dup.shape, lambda i: (0, 0, 0)),
        ],
        out_specs=pl.BlockSpec((K, BLK, 128), lambda i: (0, i, 0)),
        out_shape=jax.ShapeDtypeStruct((K, M, 128), _f32),
    )(lat_half, Wb1dup)


def _blur_gather(p_flat, blur3):
    sites = M // NWORKERS
    nch = sites // CHUNK
    CHB = 64

    @functools.partial(
        pl.kernel,
        out_type=jax.ShapeDtypeStruct((M, 128), _f32),
        mesh=_sc_mesh(),
        compiler_params=_SC_TILED,
        scratch_types=[
            pltpu.VMEM((K, CHUNK), jnp.int32),
            pltpu.VMEM((K, CHB, 128), _f32),
            pltpu.VMEM((CHB, 128), _f32),
            pltpu.SemaphoreType.DMA,
        ],
    )
    def k(p_hbm, blur_hbm, out_hbm, idx_v, g_v, acc_v, sem):
        cid = lax.axis_index("c")
        sid = lax.axis_index("s")
        wid = sid * NHALF + cid

        def chunk(c, carry):
            m0 = wid * sites + c * CHUNK
            pltpu.sync_copy(blur_hbm.at[:, wid * nch + c], idx_v)
            for w in range(2):
                descs = [
                    pltpu.async_copy(
                        p_hbm.at[idx_v.at[k_, pl.ds(w * CHB, CHB)]],
                        g_v.at[k_], sem)
                    for k_ in range(K)
                ]
                for d in descs:
                    d.wait()

                def row(i, carry2):
                    for v in range(CF // 16):
                        sl = pl.ds(v * 16, 16)
                        a = g_v[0, i, sl]
                        for k_ in range(1, K):
                            a = a + g_v[k_, i, sl]
                        acc_v[i, sl] = a
                        acc_v[i, pl.ds(CF + v * 16, 16)] = a
                    return carry2

                lax.fori_loop(0, CHB, row, 0)
                pltpu.sync_copy(acc_v, out_hbm.at[pl.ds(m0 + w * CHB, CHB)])
            return carry

        lax.fori_loop(0, nch, chunk, 0)

    return k(p_flat, blur3)


def _t5(hpre, W2dup, bb1d, bb2d):
    def body(h_ref, w_ref, b1_, b2_, out_ref):
        h = _lrelu(h_ref[...] + b1_[...])
        out_ref[...] = _lrelu(
            lax.dot_general(h, w_ref[...], (((1,), (0,)), ((), ()))) + b2_[...]
        )

    full = lambda a: pl.BlockSpec(a.shape, lambda i: (0,) * a.ndim)
    return pl.pallas_call(
        body,
        grid=(M // BLK,),
        in_specs=[
            pl.BlockSpec((BLK, 128), lambda i: (i, 0)),
            full(W2dup), full(bb1d), full(bb2d),
        ],
        out_specs=pl.BlockSpec((BLK, 128), lambda i: (i, 0)),
        out_shape=jax.ShapeDtypeStruct((M, 128), _f32),
    )(hpre, W2dup, bb1d, bb2d)


def _slice_gather(h2, offs3):
    pts = N // NWORKERS
    nch = pts // CHUNK

    @functools.partial(
        pl.kernel,
        out_type=jax.ShapeDtypeStruct((D1, N, 128), _f32),
        mesh=_sc_mesh(),
        compiler_params=_SC_TILED,
        scratch_types=[
            pltpu.VMEM((D1, CHUNK), jnp.int32),
            pltpu.VMEM((D1, CHUNK, 128), _f32),
            pltpu.SemaphoreType.DMA,
        ],
    )
    def k(h_hbm, off_hbm, out_hbm, idx_v, g_v, sem):
        cid = lax.axis_index("c")
        sid = lax.axis_index("s")
        wid = sid * NHALF + cid

        def chunk(c, carry):
            n0 = wid * pts + c * CHUNK
            pltpu.sync_copy(off_hbm.at[:, wid * nch + c], idx_v)
            descs = [
                pltpu.async_copy(h_hbm.at[idx_v.at[d]], g_v.at[d], sem)
                for d in range(D1)
            ]
            for d in descs:
                d.wait()
            for d in range(D1):
                pltpu.sync_copy(g_v.at[d], out_hbm.at[d, pl.ds(n0, CHUNK)])
            return carry

        lax.fori_loop(0, nch, chunk, 0)

    return k(h2, offs3)


def _t7(G1, G2, bary1_t, bary2_t, WoutA, WoutB, boutr):
    def body(g1_ref, g2_ref, b1_ref, b2_ref, wa_ref, wb_ref, bo_ref, out_ref):
        s1 = g1_ref[0] * b1_ref[:, 0:1]
        s2 = g2_ref[0] * b2_ref[:, 0:1]
        for d in range(1, D1):
            s1 = s1 + g1_ref[d] * b1_ref[:, d : d + 1]
            s2 = s2 + g2_ref[d] * b2_ref[:, d : d + 1]
        cdims = (((1,), (1,)), ((), ()))
        out_ref[...] = (
            lax.dot_general(wa_ref[...], s1, cdims)
            + lax.dot_general(wb_ref[...], s2, cdims)
            + bo_ref[...]
        )

    full = lambda a: pl.BlockSpec(a.shape, lambda i: (0,) * a.ndim)
    return pl.pallas_call(
        body,
        grid=(N // BLK,),
        in_specs=[
            pl.BlockSpec((D1, BLK, 128), lambda i: (0, i, 0)),
            pl.BlockSpec((D1, BLK, 128), lambda i: (0, i, 0)),
            pl.BlockSpec((BLK, D1), lambda i: (i, 0)),
            pl.BlockSpec((BLK, D1), lambda i: (i, 0)),
            full(WoutA), full(WoutB), full(boutr),
        ],
        out_specs=pl.BlockSpec((3, BLK), lambda i: (0, i)),
        out_shape=jax.ShapeDtypeStruct((3, N), _f32),
    )(G1, G2, bary1_t, bary2_t, WoutA, WoutB, boutr)


def kernel(pc1, pc2, pc1_el_minus_gr, pc2_el_minus_gr, pc1_barycentric,
           pc2_barycentric, pc1_lattice_offset, pc2_lattice_offset,
           pc1_blur_neighbors, pc2_blur_neighbors,
           W1, b1, W2, b2, W3, b3, Wb1, bb1, Wb2, bb2, Wout, bout):
    b1r, b2r, b3r = b1[None], b2[None], b3[None]
    boutr = bout[:, None]
    Wb1p = jnp.pad(jnp.transpose(Wb1, (2, 1, 0)),
                   ((0, 0), (0, 2 * CH - CIN), (0, 0)))
    Wb1dup = jnp.concatenate([Wb1p, Wb1p], axis=2)
    Wb2t = jnp.transpose(Wb2)
    W2row = jnp.concatenate([Wb2t, Wb2t], axis=1)
    W2dup = 0.5 * jnp.concatenate([W2row, W2row], axis=0)
    bb1d = jnp.concatenate([bb1, bb1])[None]
    bb2d = jnp.concatenate([bb2, bb2])[None]
    WoutA = 0.5 * jnp.concatenate([Wout[:, :CF], Wout[:, :CF]], axis=1)
    WoutB = 0.5 * jnp.concatenate([Wout[:, CF:], Wout[:, CF:]], axis=1)
    zeros_stripe = jnp.zeros((M // 16, CH), _f32)

    def one_cloud(pc, el, bary, offs, blur):
        pts_t = jnp.transpose(pc[0])
        el_t = jnp.transpose(el[0])
        bary_t = jnp.transpose(bary[0])
        idx_rows = offs[0].reshape(D1 * N // CHUNK, CHUNK)
        offs3 = offs[0].reshape(D1, N // CHUNK, CHUNK)
        blur3 = (blur[0] + (jnp.arange(K, dtype=jnp.int32) * M)[:, None]
                 ).reshape(K, M // CHUNK, CHUNK)

        w_upd = _t1(pts_t, el_t, bary_t, W1, b1r, W2, b2r, W3, b3r)
        w_upd = w_upd.reshape(NHALF, D1 * N, CH)
        lat_half = _splat(w_upd, idx_rows, zeros_stripe)
        p = _t3(lat_half, Wb1dup).reshape(K * M, 128)
        hpre = _blur_gather(p, blur3)
        h2 = _t5(hpre, W2dup, bb1d, bb2d)
        g = _slice_gather(h2, offs3)
        return g, bary_t

    g1, bary1_t = one_cloud(pc1, pc1_el_minus_gr, pc1_barycentric,
                            pc1_lattice_offset, pc1_blur_neighbors)
    g2, bary2_t = one_cloud(pc2, pc2_el_minus_gr, pc2_barycentric,
                            pc2_lattice_offset, pc2_blur_neighbors)
    flow = _t7(g1, g2, bary1_t, bary2_t, WoutA, WoutB, boutr)
    return flow[None]
